# baseline (speedup 1.0000x reference)
import functools
import math

import jax
import jax.numpy as jnp
from jax.experimental import pallas as pl
from jax.experimental.pallas import tpu as pltpu


def _attractor_body(h_hbm, e_hbm, w_ref, bias_ref, out_ref,
                    h_bufs, e_bufs, h_sems, e_sems, m_acc, s_acc,
                    *, seq_len, ch, depth, steps, batches_per_core):
    i = pl.program_id(0)
    j = pl.program_id(1)
    nj = seq_len // ch

    def start(step):
        b = batches_per_core * i + step // nj
        c = step % nj
        slot = step % depth
        pltpu.make_async_copy(
            h_hbm.at[b, pl.ds(c * ch, ch), :], h_bufs.at[slot],
            h_sems.at[slot]).start()
        pltpu.make_async_copy(
            e_hbm.at[b, pl.ds(c * ch, ch), :], e_bufs.at[slot],
            e_sems.at[slot]).start()

    @pl.when(j == 0)
    def _prologue():
        for d in range(min(depth - 1, steps)):
            start(jnp.int32(d))

    @pl.when(j + depth - 1 < steps)
    def _prefetch():
        start(j + depth - 1)

    slot = j % depth
    pltpu.make_async_copy(h_bufs.at[slot], h_bufs.at[slot],
                          h_sems.at[slot]).wait()
    pltpu.make_async_copy(e_bufs.at[slot], e_bufs.at[slot],
                          e_sems.at[slot]).wait()

    jj = j % nj
    ti = jj * ch + jax.lax.broadcasted_iota(jnp.int32, (ch, 1), 0)
    decay = jnp.exp((seq_len - 1.0 - ti.astype(jnp.float32))
                    * (-math.pi / seq_len))
    h = h_bufs[slot]
    e = e_bufs[slot]
    hw = h * decay
    contrib = jax.lax.dot_general(
        hw, e, (((0,), (0,)), ((), ())),
        preferred_element_type=jnp.float32,
    )
    s_contrib = jnp.sum(decay * e, axis=0, keepdims=True)

    @pl.when(jj == 0)
    def _init():
        m_acc[...] = contrib
        s_acc[...] = s_contrib

    @pl.when(jj != 0)
    def _accum():
        m_acc[...] += contrib
        s_acc[...] += s_contrib

    @pl.when(jj == nj - 1)
    def _finish():
        out_ref[0] = jax.lax.dot_general(
            w_ref[...], m_acc[...], (((1,), (0,)), ((), ())),
            preferred_element_type=jnp.float32,
        ) + bias_ref[...] * s_acc[...]


def kernel(hidden_states, positional_encodings, W, b):
    bsz, seq_len, d_model = hidden_states.shape
    d_state = W.shape[0]
    ch = 2048
    depth = 4
    n_cores = 2 if bsz % 2 == 0 else 1
    batches_per_core = bsz // n_cores
    nj = seq_len // ch
    steps = batches_per_core * nj
    assert seq_len % ch == 0
    bias_col = b.reshape(d_state, 1)

    body = functools.partial(
        _attractor_body, seq_len=seq_len, ch=ch, depth=depth, steps=steps,
        batches_per_core=batches_per_core)

    return pl.pallas_call(
        body,
        out_shape=jax.ShapeDtypeStruct((bsz, d_state, d_model), jnp.float32),
        grid=(n_cores, steps),
        in_specs=[
            pl.BlockSpec(memory_space=pl.ANY),
            pl.BlockSpec(memory_space=pl.ANY),
            pl.BlockSpec((d_state, d_model), lambda i, j: (0, 0)),
            pl.BlockSpec((d_state, 1), lambda i, j: (0, 0)),
        ],
        out_specs=pl.BlockSpec(
            (1, d_state, d_model),
            lambda i, j, _bpc=batches_per_core, _nj=nj:
                (_bpc * i + j // _nj, 0, 0)),
        scratch_shapes=[
            pltpu.VMEM((depth, ch, d_model), jnp.float32),
            pltpu.VMEM((depth, ch, d_model), jnp.float32),
            pltpu.SemaphoreType.DMA((depth,)),
            pltpu.SemaphoreType.DMA((depth,)),
            pltpu.VMEM((d_model, d_model), jnp.float32),
            pltpu.VMEM((1, d_model), jnp.float32),
        ],
        compiler_params=pltpu.CompilerParams(
            dimension_semantics=("parallel", "arbitrary"),
        ),
        name="attractor_state",
    )(hidden_states, positional_encodings, W, bias_col)


# speedup vs baseline: 1.0594x; 1.0594x over previous
import functools
import math

import jax
import jax.numpy as jnp
from jax.experimental import pallas as pl
from jax.experimental.pallas import tpu as pltpu


def _attractor_body(h_ref, e_ref, w_ref, bias_ref, out_ref, m_acc, s_acc,
                    *, seq_len, chunk):
    j = pl.program_id(1)
    nj = pl.num_programs(1)
    ti = j * chunk + jax.lax.broadcasted_iota(jnp.int32, (chunk, 1), 0)
    decay = jnp.exp((seq_len - 1.0 - ti.astype(jnp.float32))
                    * (-math.pi / seq_len))
    hw = h_ref[0] * decay
    e = e_ref[0]
    contrib = jax.lax.dot_general(
        hw, e, (((0,), (0,)), ((), ())),
        preferred_element_type=jnp.float32,
    )
    s_contrib = jax.lax.dot_general(
        decay, e, (((0,), (0,)), ((), ())),
        preferred_element_type=jnp.float32,
    )

    @pl.when(j == 0)
    def _init():
        m_acc[...] = contrib
        s_acc[...] = s_contrib

    @pl.when(j != 0)
    def _accum():
        m_acc[...] += contrib
        s_acc[...] += s_contrib

    @pl.when(j == nj - 1)
    def _finish():
        out_ref[0] = jax.lax.dot_general(
            w_ref[...], m_acc[...], (((1,), (0,)), ((), ())),
            preferred_element_type=jnp.float32,
        ) + bias_ref[...] * s_acc[...]


def kernel(hidden_states, positional_encodings, W, b):
    bsz, seq_len, d_model = hidden_states.shape
    d_state = W.shape[0]
    chunk = 4096
    assert seq_len % chunk == 0
    bias_col = b.reshape(d_state, 1)

    body = functools.partial(_attractor_body, seq_len=seq_len, chunk=chunk)

    return pl.pallas_call(
        body,
        out_shape=jax.ShapeDtypeStruct((bsz, d_state, d_model), jnp.float32),
        grid=(bsz, seq_len // chunk),
        in_specs=[
            pl.BlockSpec((1, chunk, d_model), lambda i, j: (i, j, 0)),
            pl.BlockSpec((1, chunk, d_model), lambda i, j: (i, j, 0)),
            pl.BlockSpec((d_state, d_model), lambda i, j: (0, 0)),
            pl.BlockSpec((d_state, 1), lambda i, j: (0, 0)),
        ],
        out_specs=pl.BlockSpec((1, d_state, d_model), lambda i, j: (i, 0, 0)),
        scratch_shapes=[
            pltpu.VMEM((d_model, d_model), jnp.float32),
            pltpu.VMEM((1, d_model), jnp.float32),
        ],
        compiler_params=pltpu.CompilerParams(
            dimension_semantics=("parallel", "arbitrary"),
        ),
        name="attractor_state",
    )(hidden_states, positional_encodings, W, bias_col)


# speedup vs baseline: 1.0827x; 1.0221x over previous
import functools
import math

import jax
import jax.numpy as jnp
from jax.experimental import pallas as pl
from jax.experimental.pallas import tpu as pltpu


def _attractor_body(h_ref, e_ref, w_ref, bias_ref, out_ref, m_acc, s_acc,
                    *, seq_len, chunk):
    j = pl.program_id(1)
    nj = pl.num_programs(1)
    ti = j * chunk + jax.lax.broadcasted_iota(jnp.int32, (chunk, 1), 0)
    decay = jnp.exp((seq_len - 1.0 - ti.astype(jnp.float32))
                    * (-math.pi / seq_len))
    hw = h_ref[0] * decay
    e = e_ref[0]
    contrib = jax.lax.dot_general(
        hw, e, (((0,), (0,)), ((), ())),
        preferred_element_type=jnp.float32,
    )
    s_contrib = jnp.sum(decay * e, axis=0, keepdims=True)

    @pl.when(j == 0)
    def _init():
        m_acc[...] = contrib
        s_acc[...] = s_contrib

    @pl.when(j != 0)
    def _accum():
        m_acc[...] += contrib
        s_acc[...] += s_contrib

    @pl.when(j == nj - 1)
    def _finish():
        out_ref[0] = jax.lax.dot_general(
            w_ref[...], m_acc[...], (((1,), (0,)), ((), ())),
            preferred_element_type=jnp.float32,
        ) + bias_ref[...] * s_acc[...]


def kernel(hidden_states, positional_encodings, W, b):
    bsz, seq_len, d_model = hidden_states.shape
    d_state = W.shape[0]
    chunk = 4096
    assert seq_len % chunk == 0
    bias_col = b.reshape(d_state, 1)

    body = functools.partial(_attractor_body, seq_len=seq_len, chunk=chunk)

    return pl.pallas_call(
        body,
        out_shape=jax.ShapeDtypeStruct((bsz, d_state, d_model), jnp.float32),
        grid=(bsz, seq_len // chunk),
        in_specs=[
            pl.BlockSpec((1, chunk, d_model), lambda i, j: (i, j, 0)),
            pl.BlockSpec((1, chunk, d_model), lambda i, j: (i, j, 0)),
            pl.BlockSpec((d_state, d_model), lambda i, j: (0, 0)),
            pl.BlockSpec((d_state, 1), lambda i, j: (0, 0)),
        ],
        out_specs=pl.BlockSpec((1, d_state, d_model), lambda i, j: (i, 0, 0)),
        scratch_shapes=[
            pltpu.VMEM((d_model, d_model), jnp.float32),
            pltpu.VMEM((1, d_model), jnp.float32),
        ],
        compiler_params=pltpu.CompilerParams(
            dimension_semantics=("parallel", "arbitrary"),
        ),
        name="attractor_state",
    )(hidden_states, positional_encodings, W, bias_col)
